# baseline (speedup 1.0000x reference)
import functools

import jax
import jax.numpy as jnp
from jax import lax
from jax.experimental import pallas as pl
from jax.experimental.pallas import tpu as pltpu
from jax.experimental.pallas import tpu_sc as plsc

N = 10000
T = 8
D_IN = 128
H = 128
D_OUT = 128
E = 320000

NC = 2
NS = 16
NW = NC * NS
KC = 128
EPW = E // NW
EPWP = 10240
CPW = EPWP // KC
NP = 10240
RPT = NP // NS
BN = 1000


def _sc_mesh():
    return plsc.VectorSubcoreMesh(core_axis_name="c", subcore_axis_name="s")


Q = 20
NQ = 4


def _scat_body(hws_hbm, eidx_hbm, zeros_hbm, out_hbm,
               ib0, ib1, gb0, gb1, iq0, iq1, gs0, gs1, ps0, ps1, acc_ref):
    cid = lax.axis_index("c")
    sid = lax.axis_index("s")
    wid = cid * NS + sid
    pltpu.sync_copy(zeros_hbm.at[pl.ds(sid * RPT, RPT)],
                    acc_ref.at[pl.ds(sid * RPT, RPT)])
    ibufs = (ib0, ib1)
    isems = (iq0, iq1)
    handles = [
        pltpu.async_copy(eidx_hbm.at[wid, pl.ds(0, Q)], ib0, iq0),
        pltpu.async_copy(eidx_hbm.at[wid, pl.ds(Q, Q)], ib1, iq1),
    ]
    plsc.subcore_barrier()

    for qq in range(NQ):
        ib = ibufs[qq % 2]
        handles[qq].wait()

        def inner(i, carry, _ib=ib):
            j = 2 * i
            g0 = pltpu.async_copy(hws_hbm.at[_ib.at[j, 0]], gb0, gs0)
            g1 = pltpu.async_copy(hws_hbm.at[_ib.at[j + 1, 0]], gb1, gs1)
            g0.wait()
            s0 = pltpu.async_copy(gb0, acc_ref.at[_ib.at[j, 1]], ps0, add=True)
            g1.wait()
            s1 = pltpu.async_copy(gb1, acc_ref.at[_ib.at[j + 1, 1]], ps1,
                                  add=True)
            s0.wait()
            s1.wait()
            return carry

        lax.fori_loop(0, Q // 2, inner, 0)
        if qq + 2 < NQ:
            handles.append(pltpu.async_copy(
                eidx_hbm.at[wid, pl.ds((qq + 2) * Q, Q)], ib, isems[qq % 2]))

    plsc.subcore_barrier()
    pltpu.sync_copy(acc_ref.at[pl.ds(sid * RPT, RPT)],
                    out_hbm.at[cid, pl.ds(sid * RPT, RPT)])


def _make_scat_kernel():
    return pl.kernel(
        _scat_body,
        out_type=jax.ShapeDtypeStruct((NC, NP, H), jnp.float32),
        mesh=_sc_mesh(),
        scratch_types=[
            pltpu.VMEM((Q, 2, KC), jnp.int32),
            pltpu.VMEM((Q, 2, KC), jnp.int32),
            pltpu.VMEM((KC, H), jnp.float32),
            pltpu.VMEM((KC, H), jnp.float32),
            pltpu.SemaphoreType.DMA,
            pltpu.SemaphoreType.DMA,
            pltpu.SemaphoreType.DMA,
            pltpu.SemaphoreType.DMA,
            pltpu.SemaphoreType.DMA,
            pltpu.SemaphoreType.DMA,
            pltpu.VMEM_SHARED((NP, H), jnp.float32),
        ],
    )


def _deg_body(eidx_hbm, ones_hbm, zeros_hbm, out_hbm, idxv, onesv, acc_ref):
    cid = lax.axis_index("c")
    sid = lax.axis_index("s")
    wid = cid * NS + sid
    pltpu.sync_copy(eidx_hbm.at[wid], idxv)
    pltpu.sync_copy(ones_hbm.at[pl.ds(0, KC)], onesv)
    pltpu.sync_copy(zeros_hbm.at[pl.ds(sid * RPT, RPT)],
                    acc_ref.at[pl.ds(sid * RPT, RPT)])
    plsc.subcore_barrier()

    def body(j, carry):
        pltpu.sync_copy(onesv, acc_ref.at[idxv.at[j, 1]], add=True)
        return carry

    lax.fori_loop(0, CPW, body, 0)
    plsc.subcore_barrier()
    pltpu.sync_copy(acc_ref.at[pl.ds(sid * RPT, RPT)],
                    out_hbm.at[cid, pl.ds(sid * RPT, RPT)])


def _make_deg_kernel():
    return pl.kernel(
        _deg_body,
        out_type=jax.ShapeDtypeStruct((NC, NP, H), jnp.float32),
        mesh=_sc_mesh(),
        scratch_types=[
            pltpu.VMEM((CPW, 2, KC), jnp.int32),
            pltpu.VMEM((KC, H), jnp.float32),
            pltpu.VMEM_SHARED((NP, H), jnp.float32),
        ],
    )


def _dinv_from_deg(deg):
    d = deg[0, :, 0:1] + deg[1, :, 0:1] + 1.0
    return lax.rsqrt(d)


def _gru_hws(gi, hprev, dinv, whh, bhh, wg):
    dn = (((1,), (1,)), ((), ()))
    gh = lax.dot_general(hprev, whh, dn, preferred_element_type=jnp.float32) + bhh
    r = jax.nn.sigmoid(gi[:, :H] + gh[:, :H])
    z = jax.nn.sigmoid(gi[:, H:2 * H] + gh[:, H:2 * H])
    cand = jnp.tanh(gi[:, 2 * H:] + r * gh[:, 2 * H:])
    h = (1.0 - z) * cand + z * hprev
    hw = lax.dot_general(h, wg, dn, preferred_element_type=jnp.float32)
    return dinv * hw


_W_SPECS = [
    pl.BlockSpec((3 * H, H), lambda i: (0, 0)),
    pl.BlockSpec((1, 3 * H), lambda i: (0, 0)),
    pl.BlockSpec((H, H), lambda i: (0, 0)),
    pl.BlockSpec((1, H), lambda i: (0, 0)),
]


def _make_gi_kernel(t):
    def body(x_ref, we, be, wih, bih, out_ref):
        dn = (((1,), (1,)), ((), ()))
        inp = lax.dot_general(x_ref[0], we[...], dn,
                              preferred_element_type=jnp.float32) + be[...]
        out_ref[...] = lax.dot_general(
            inp, wih[...], dn, preferred_element_type=jnp.float32) + bih[...]

    in_specs = [
        pl.BlockSpec((1, BN, D_IN), lambda i, _t=t: (_t, i, 0)),
        pl.BlockSpec((H, D_IN), lambda i: (0, 0)),
        pl.BlockSpec((1, H), lambda i: (0, 0)),
        pl.BlockSpec((3 * H, H), lambda i: (0, 0)),
        pl.BlockSpec((1, 3 * H), lambda i: (0, 0)),
    ]
    return pl.pallas_call(
        body,
        grid=(N // BN,),
        in_specs=in_specs,
        out_specs=pl.BlockSpec((BN, 3 * H), lambda i: (i, 0)),
        out_shape=jax.ShapeDtypeStruct((N, 3 * H), jnp.float32),
    )


def _make_step0_kernel():
    def body(gi_ref, deg_ref, whh, bhh, wg, bg, out_ref):
        dinv = _dinv_from_deg(deg_ref[...])
        hprev = jnp.zeros((BN, H), jnp.float32)
        out_ref[...] = _gru_hws(gi_ref[...], hprev, dinv,
                                whh[...], bhh[...], wg[...])

    in_specs = [
        pl.BlockSpec((BN, 3 * H), lambda i: (i, 0)),
        pl.BlockSpec((NC, BN, H), lambda i: (0, i, 0)),
    ] + _W_SPECS
    return pl.pallas_call(
        body,
        grid=(N // BN,),
        in_specs=in_specs,
        out_specs=pl.BlockSpec((BN, H), lambda i: (i, 0)),
        out_shape=jax.ShapeDtypeStruct((N, H), jnp.float32),
    )


def _make_step_kernel(t):
    def body(gi_ref, deg_ref, part_ref, hwsp_ref, whh, bhh,
             wg, bg, out_ref):
        dinv = _dinv_from_deg(deg_ref[...])
        p = part_ref[...]
        hprev = dinv * (p[0] + p[1] + hwsp_ref[...]) + bg[...]
        out_ref[...] = _gru_hws(gi_ref[...], hprev, dinv,
                                whh[...], bhh[...], wg[...])

    in_specs = [
        pl.BlockSpec((BN, 3 * H), lambda i: (i, 0)),
        pl.BlockSpec((NC, BN, H), lambda i: (0, i, 0)),
        pl.BlockSpec((NC, BN, H), lambda i: (0, i, 0)),
        pl.BlockSpec((BN, H), lambda i: (i, 0)),
    ] + _W_SPECS
    return pl.pallas_call(
        body,
        grid=(N // BN,),
        in_specs=in_specs,
        out_specs=pl.BlockSpec((BN, H), lambda i: (i, 0)),
        out_shape=jax.ShapeDtypeStruct((N, H), jnp.float32),
    )


def _make_final_kernel():
    def body(deg_ref, part_ref, hwsp_ref, bg, wfc, bfc, out_ref):
        dinv = _dinv_from_deg(deg_ref[...])
        p = part_ref[...]
        h = dinv * (p[0] + p[1] + hwsp_ref[...]) + bg[...]
        dn = (((1,), (1,)), ((), ()))
        out_ref[...] = lax.dot_general(
            h, wfc[...], dn, preferred_element_type=jnp.float32) + bfc[...]

    in_specs = [
        pl.BlockSpec((NC, BN, H), lambda i: (0, i, 0)),
        pl.BlockSpec((NC, BN, H), lambda i: (0, i, 0)),
        pl.BlockSpec((BN, H), lambda i: (i, 0)),
        pl.BlockSpec((1, H), lambda i: (0, 0)),
        pl.BlockSpec((D_OUT, H), lambda i: (0, 0)),
        pl.BlockSpec((1, D_OUT), lambda i: (0, 0)),
    ]
    return pl.pallas_call(
        body,
        grid=(N // BN,),
        in_specs=in_specs,
        out_specs=pl.BlockSpec((BN, D_OUT), lambda i: (i, 0)),
        out_shape=jax.ShapeDtypeStruct((N, D_OUT), jnp.float32),
    )


def kernel(x, edge_index, W_emb, b_emb, W_ih, W_hh, b_ih, b_hh, W_gcn, b_gcn,
           W_fc, b_fc):
    pad = EPWP - EPW
    padr = jnp.broadcast_to(jnp.arange(pad, dtype=jnp.int32), (NW, pad))
    padc = jnp.broadcast_to(N + jnp.arange(pad, dtype=jnp.int32), (NW, pad))
    rw = jnp.concatenate([edge_index[0].reshape(NW, EPW), padr], axis=1)
    cw = jnp.concatenate([edge_index[1].reshape(NW, EPW), padc], axis=1)
    eidx = jnp.stack([rw.reshape(NW, CPW, KC), cw.reshape(NW, CPW, KC)],
                     axis=2)
    xT = jnp.transpose(x, (1, 0, 2))
    zeros_nh = jnp.zeros((NP, H), jnp.float32)
    ones_nh = jnp.ones((N, H), jnp.float32)
    be = b_emb.reshape(1, H)
    bih = b_ih.reshape(1, 3 * H)
    bhh = b_hh.reshape(1, 3 * H)
    bg = b_gcn.reshape(1, H)
    bfc = b_fc.reshape(1, D_OUT)

    scat = _make_scat_kernel()
    deg = _make_deg_kernel()(eidx, ones_nh, zeros_nh)

    gis = [_make_gi_kernel(t)(xT, W_emb, be, W_ih, bih) for t in range(T)]
    hws = _make_step0_kernel()(gis[0], deg, W_hh, bhh, W_gcn, bg)
    for t in range(T):
        part = scat(hws, eidx, zeros_nh)
        if t < T - 1:
            hws = _make_step_kernel(t + 1)(
                gis[t + 1], deg, part, hws, W_hh, bhh, W_gcn, bg)
    return _make_final_kernel()(deg, part, hws, bg, W_fc, bfc)


# speedup vs baseline: 1.0037x; 1.0037x over previous
import functools

import jax
import jax.numpy as jnp
from jax import lax
from jax.experimental import pallas as pl
from jax.experimental.pallas import tpu as pltpu
from jax.experimental.pallas import tpu_sc as plsc

N = 10000
T = 8
D_IN = 128
H = 128
D_OUT = 128
E = 320000

NC = 2
NS = 16
NW = NC * NS
KC = 128
EPW = E // NW
EPWP = 10240
CPW = EPWP // KC
NP = 10240
RPT = NP // NS
BN = 1000


def _sc_mesh():
    return plsc.VectorSubcoreMesh(core_axis_name="c", subcore_axis_name="s")


Q = 20
NQ = 4


def _scat_body(hws_hbm, eidx_hbm, zeros_hbm, out_hbm,
               ib0, ib1, gb0, gb1, iq0, iq1, gs0, gs1, ps0, ps1, acc_ref):
    cid = lax.axis_index("c")
    sid = lax.axis_index("s")
    wid = cid * NS + sid
    pltpu.sync_copy(zeros_hbm.at[pl.ds(sid * RPT, RPT)],
                    acc_ref.at[pl.ds(sid * RPT, RPT)])
    ibufs = (ib0, ib1)
    isems = (iq0, iq1)
    handles = [
        pltpu.async_copy(eidx_hbm.at[wid, pl.ds(0, Q)], ib0, iq0),
        pltpu.async_copy(eidx_hbm.at[wid, pl.ds(Q, Q)], ib1, iq1),
    ]
    plsc.subcore_barrier()

    for qq in range(NQ):
        ib = ibufs[qq % 2]
        handles[qq].wait()

        def inner(i, carry, _ib=ib):
            j = 2 * i
            g0 = pltpu.async_copy(hws_hbm.at[_ib.at[j, 0]], gb0, gs0)
            g1 = pltpu.async_copy(hws_hbm.at[_ib.at[j + 1, 0]], gb1, gs1)
            g0.wait()
            s0 = pltpu.async_copy(gb0, acc_ref.at[_ib.at[j, 1]], ps0, add=True)
            g1.wait()
            s1 = pltpu.async_copy(gb1, acc_ref.at[_ib.at[j + 1, 1]], ps1,
                                  add=True)
            s0.wait()
            s1.wait()
            return carry

        lax.fori_loop(0, Q // 2, inner, 0)
        if qq + 2 < NQ:
            handles.append(pltpu.async_copy(
                eidx_hbm.at[wid, pl.ds((qq + 2) * Q, Q)], ib, isems[qq % 2]))

    plsc.subcore_barrier()
    pltpu.sync_copy(acc_ref.at[pl.ds(sid * RPT, RPT)],
                    out_hbm.at[cid, pl.ds(sid * RPT, RPT)])


def _make_scat_kernel():
    return pl.kernel(
        _scat_body,
        out_type=jax.ShapeDtypeStruct((NC, NP, H), jnp.float32),
        mesh=_sc_mesh(),
        scratch_types=[
            pltpu.VMEM((Q, 2, KC), jnp.int32),
            pltpu.VMEM((Q, 2, KC), jnp.int32),
            pltpu.VMEM((KC, H), jnp.float32),
            pltpu.VMEM((KC, H), jnp.float32),
            pltpu.SemaphoreType.DMA,
            pltpu.SemaphoreType.DMA,
            pltpu.SemaphoreType.DMA,
            pltpu.SemaphoreType.DMA,
            pltpu.SemaphoreType.DMA,
            pltpu.SemaphoreType.DMA,
            pltpu.VMEM_SHARED((NP, H), jnp.float32),
        ],
    )


def _deg_body(eidx_hbm, ones_hbm, zeros_hbm, out_hbm, idxv, onesv, d0, d1,
              acc_ref):
    cid = lax.axis_index("c")
    sid = lax.axis_index("s")
    wid = cid * NS + sid
    pltpu.sync_copy(eidx_hbm.at[wid], idxv)
    pltpu.sync_copy(ones_hbm.at[pl.ds(0, KC)], onesv)
    pltpu.sync_copy(zeros_hbm.at[pl.ds(sid * RPT, RPT)],
                    acc_ref.at[pl.ds(sid * RPT, RPT)])
    plsc.subcore_barrier()

    def body(q, carry):
        j = 2 * q
        s0 = pltpu.async_copy(onesv, acc_ref.at[idxv.at[j, 1]], d0, add=True)
        s1 = pltpu.async_copy(onesv, acc_ref.at[idxv.at[j + 1, 1]], d1,
                              add=True)
        s0.wait()
        s1.wait()
        return carry

    lax.fori_loop(0, CPW // 2, body, 0)
    plsc.subcore_barrier()
    pltpu.sync_copy(acc_ref.at[pl.ds(sid * RPT, RPT)],
                    out_hbm.at[cid, pl.ds(sid * RPT, RPT)])


def _make_deg_kernel():
    return pl.kernel(
        _deg_body,
        out_type=jax.ShapeDtypeStruct((NC, NP, H), jnp.float32),
        mesh=_sc_mesh(),
        scratch_types=[
            pltpu.VMEM((CPW, 2, KC), jnp.int32),
            pltpu.VMEM((KC, H), jnp.float32),
            pltpu.SemaphoreType.DMA,
            pltpu.SemaphoreType.DMA,
            pltpu.VMEM_SHARED((NP, H), jnp.float32),
        ],
    )


def _dinv_from_deg(deg):
    d = deg[0, :, 0:1] + deg[1, :, 0:1] + 1.0
    return lax.rsqrt(d)


def _gru_hws(x_blk, hprev, dinv, we, be, wih, bih, whh, bhh, wg):
    dn = (((1,), (1,)), ((), ()))
    inp = lax.dot_general(x_blk, we, dn, preferred_element_type=jnp.float32) + be
    gi = lax.dot_general(inp, wih, dn, preferred_element_type=jnp.float32) + bih
    gh = lax.dot_general(hprev, whh, dn, preferred_element_type=jnp.float32) + bhh
    r = jax.nn.sigmoid(gi[:, :H] + gh[:, :H])
    z = jax.nn.sigmoid(gi[:, H:2 * H] + gh[:, H:2 * H])
    cand = jnp.tanh(gi[:, 2 * H:] + r * gh[:, 2 * H:])
    h = (1.0 - z) * cand + z * hprev
    hw = lax.dot_general(h, wg, dn, preferred_element_type=jnp.float32)
    return dinv * hw


_W_SPECS = [
    pl.BlockSpec((H, D_IN), lambda i: (0, 0)),
    pl.BlockSpec((1, H), lambda i: (0, 0)),
    pl.BlockSpec((3 * H, H), lambda i: (0, 0)),
    pl.BlockSpec((1, 3 * H), lambda i: (0, 0)),
    pl.BlockSpec((3 * H, H), lambda i: (0, 0)),
    pl.BlockSpec((1, 3 * H), lambda i: (0, 0)),
    pl.BlockSpec((H, H), lambda i: (0, 0)),
    pl.BlockSpec((1, H), lambda i: (0, 0)),
]


def _make_step0_kernel():
    def body(x_ref, deg_ref, we, be, wih, bih, whh, bhh, wg, bg, out_ref):
        dinv = _dinv_from_deg(deg_ref[...])
        hprev = jnp.zeros((BN, H), jnp.float32)
        out_ref[...] = _gru_hws(x_ref[0], hprev, dinv, we[...], be[...],
                                wih[...], bih[...], whh[...], bhh[...], wg[...])

    in_specs = [
        pl.BlockSpec((1, BN, D_IN), lambda i: (0, i, 0)),
        pl.BlockSpec((NC, BN, H), lambda i: (0, i, 0)),
    ] + _W_SPECS
    return pl.pallas_call(
        body,
        grid=(N // BN,),
        in_specs=in_specs,
        out_specs=pl.BlockSpec((BN, H), lambda i: (i, 0)),
        out_shape=jax.ShapeDtypeStruct((N, H), jnp.float32),
    )


def _make_step_kernel(t):
    def body(x_ref, deg_ref, part_ref, hwsp_ref, we, be, wih, bih, whh, bhh,
             wg, bg, out_ref):
        dinv = _dinv_from_deg(deg_ref[...])
        p = part_ref[...]
        hprev = dinv * (p[0] + p[1] + hwsp_ref[...]) + bg[...]
        out_ref[...] = _gru_hws(x_ref[0], hprev, dinv, we[...], be[...],
                                wih[...], bih[...], whh[...], bhh[...], wg[...])

    in_specs = [
        pl.BlockSpec((1, BN, D_IN), lambda i, _t=t: (_t, i, 0)),
        pl.BlockSpec((NC, BN, H), lambda i: (0, i, 0)),
        pl.BlockSpec((NC, BN, H), lambda i: (0, i, 0)),
        pl.BlockSpec((BN, H), lambda i: (i, 0)),
    ] + _W_SPECS
    return pl.pallas_call(
        body,
        grid=(N // BN,),
        in_specs=in_specs,
        out_specs=pl.BlockSpec((BN, H), lambda i: (i, 0)),
        out_shape=jax.ShapeDtypeStruct((N, H), jnp.float32),
    )


def _make_final_kernel():
    def body(deg_ref, part_ref, hwsp_ref, bg, wfc, bfc, out_ref):
        dinv = _dinv_from_deg(deg_ref[...])
        p = part_ref[...]
        h = dinv * (p[0] + p[1] + hwsp_ref[...]) + bg[...]
        dn = (((1,), (1,)), ((), ()))
        out_ref[...] = lax.dot_general(
            h, wfc[...], dn, preferred_element_type=jnp.float32) + bfc[...]

    in_specs = [
        pl.BlockSpec((NC, BN, H), lambda i: (0, i, 0)),
        pl.BlockSpec((NC, BN, H), lambda i: (0, i, 0)),
        pl.BlockSpec((BN, H), lambda i: (i, 0)),
        pl.BlockSpec((1, H), lambda i: (0, 0)),
        pl.BlockSpec((D_OUT, H), lambda i: (0, 0)),
        pl.BlockSpec((1, D_OUT), lambda i: (0, 0)),
    ]
    return pl.pallas_call(
        body,
        grid=(N // BN,),
        in_specs=in_specs,
        out_specs=pl.BlockSpec((BN, D_OUT), lambda i: (i, 0)),
        out_shape=jax.ShapeDtypeStruct((N, D_OUT), jnp.float32),
    )


def kernel(x, edge_index, W_emb, b_emb, W_ih, W_hh, b_ih, b_hh, W_gcn, b_gcn,
           W_fc, b_fc):
    pad = EPWP - EPW
    padr = jnp.broadcast_to(jnp.arange(pad, dtype=jnp.int32), (NW, pad))
    padc = jnp.broadcast_to(N + jnp.arange(pad, dtype=jnp.int32), (NW, pad))
    rw = jnp.concatenate([edge_index[0].reshape(NW, EPW), padr], axis=1)
    cw = jnp.concatenate([edge_index[1].reshape(NW, EPW), padc], axis=1)
    eidx = jnp.stack([rw.reshape(NW, CPW, KC), cw.reshape(NW, CPW, KC)],
                     axis=2)
    xT = jnp.transpose(x, (1, 0, 2))
    zeros_nh = jnp.zeros((NP, H), jnp.float32)
    ones_nh = jnp.ones((N, H), jnp.float32)
    be = b_emb.reshape(1, H)
    bih = b_ih.reshape(1, 3 * H)
    bhh = b_hh.reshape(1, 3 * H)
    bg = b_gcn.reshape(1, H)
    bfc = b_fc.reshape(1, D_OUT)

    scat = _make_scat_kernel()
    deg = _make_deg_kernel()(eidx, ones_nh, zeros_nh)

    hws = _make_step0_kernel()(
        xT, deg, W_emb, be, W_ih, bih, W_hh, bhh, W_gcn, bg)
    for t in range(T):
        part = scat(hws, eidx, zeros_nh)
        if t < T - 1:
            hws = _make_step_kernel(t + 1)(
                xT, deg, part, hws, W_emb, be, W_ih, bih, W_hh, bhh, W_gcn, bg)
    return _make_final_kernel()(deg, part, hws, bg, W_fc, bfc)


# speedup vs baseline: 1.0064x; 1.0026x over previous
import functools

import jax
import jax.numpy as jnp
from jax import lax
from jax.experimental import pallas as pl
from jax.experimental.pallas import tpu as pltpu
from jax.experimental.pallas import tpu_sc as plsc

N = 10000
T = 8
D_IN = 128
H = 128
D_OUT = 128
E = 320000

NC = 2
NS = 16
NW = NC * NS
KC = 128
EPW = E // NW
EPWP = 10240
CPW = EPWP // KC
NP = 10240
RPT = NP // NS
BN = 1000


def _sc_mesh():
    return plsc.VectorSubcoreMesh(core_axis_name="c", subcore_axis_name="s")


Q = 20
NQ = 4


def _scat_body(hws_hbm, eidx_hbm, zeros_hbm, out_hbm,
               ib0, ib1, gb0, gb1, iq0, iq1, gs0, gs1, ps0, ps1, acc_ref):
    cid = lax.axis_index("c")
    sid = lax.axis_index("s")
    wid = cid * NS + sid
    pltpu.sync_copy(zeros_hbm.at[pl.ds(sid * RPT, RPT)],
                    acc_ref.at[pl.ds(sid * RPT, RPT)])
    ibufs = (ib0, ib1)
    isems = (iq0, iq1)
    handles = [
        pltpu.async_copy(eidx_hbm.at[wid, pl.ds(0, Q)], ib0, iq0),
        pltpu.async_copy(eidx_hbm.at[wid, pl.ds(Q, Q)], ib1, iq1),
    ]
    plsc.subcore_barrier()

    def pair(j, drain, _ib):
        if drain:
            pltpu.make_async_copy(zeros_hbm.at[pl.ds(0, KC)], gb0, ps0).wait()
            pltpu.make_async_copy(zeros_hbm.at[pl.ds(0, KC)], gb1, ps1).wait()
        g0 = pltpu.async_copy(hws_hbm.at[_ib.at[j, 0]], gb0, gs0)
        g1 = pltpu.async_copy(hws_hbm.at[_ib.at[j + 1, 0]], gb1, gs1)
        g0.wait()
        pltpu.async_copy(gb0, acc_ref.at[_ib.at[j, 1]], ps0, add=True)
        g1.wait()
        pltpu.async_copy(gb1, acc_ref.at[_ib.at[j + 1, 1]], ps1, add=True)

    for qq in range(NQ):
        ib = ibufs[qq % 2]
        handles[qq].wait()
        pair(0, qq > 0, ib)

        def inner(i, carry, _ib=ib):
            pair(2 * i, True, _ib)
            return carry

        lax.fori_loop(1, Q // 2, inner, 0)
        if qq + 2 < NQ:
            handles.append(pltpu.async_copy(
                eidx_hbm.at[wid, pl.ds((qq + 2) * Q, Q)], ib, isems[qq % 2]))
    pltpu.make_async_copy(zeros_hbm.at[pl.ds(0, KC)], gb0, ps0).wait()
    pltpu.make_async_copy(zeros_hbm.at[pl.ds(0, KC)], gb1, ps1).wait()

    plsc.subcore_barrier()
    pltpu.sync_copy(acc_ref.at[pl.ds(sid * RPT, RPT)],
                    out_hbm.at[cid, pl.ds(sid * RPT, RPT)])


def _make_scat_kernel():
    return pl.kernel(
        _scat_body,
        out_type=jax.ShapeDtypeStruct((NC, NP, H), jnp.float32),
        mesh=_sc_mesh(),
        scratch_types=[
            pltpu.VMEM((Q, 2, KC), jnp.int32),
            pltpu.VMEM((Q, 2, KC), jnp.int32),
            pltpu.VMEM((KC, H), jnp.float32),
            pltpu.VMEM((KC, H), jnp.float32),
            pltpu.SemaphoreType.DMA,
            pltpu.SemaphoreType.DMA,
            pltpu.SemaphoreType.DMA,
            pltpu.SemaphoreType.DMA,
            pltpu.SemaphoreType.DMA,
            pltpu.SemaphoreType.DMA,
            pltpu.VMEM_SHARED((NP, H), jnp.float32),
        ],
    )


def _deg_body(eidx_hbm, ones_hbm, zeros_hbm, out_hbm, idxv, onesv, d0, d1,
              acc_ref):
    cid = lax.axis_index("c")
    sid = lax.axis_index("s")
    wid = cid * NS + sid
    pltpu.sync_copy(eidx_hbm.at[wid], idxv)
    pltpu.sync_copy(ones_hbm.at[pl.ds(0, KC)], onesv)
    pltpu.sync_copy(zeros_hbm.at[pl.ds(sid * RPT, RPT)],
                    acc_ref.at[pl.ds(sid * RPT, RPT)])
    plsc.subcore_barrier()

    def body(q, carry):
        j = 2 * q
        s0 = pltpu.async_copy(onesv, acc_ref.at[idxv.at[j, 1]], d0, add=True)
        s1 = pltpu.async_copy(onesv, acc_ref.at[idxv.at[j + 1, 1]], d1,
                              add=True)
        s0.wait()
        s1.wait()
        return carry

    lax.fori_loop(0, CPW // 2, body, 0)
    plsc.subcore_barrier()
    pltpu.sync_copy(acc_ref.at[pl.ds(sid * RPT, RPT)],
                    out_hbm.at[cid, pl.ds(sid * RPT, RPT)])


def _make_deg_kernel():
    return pl.kernel(
        _deg_body,
        out_type=jax.ShapeDtypeStruct((NC, NP, H), jnp.float32),
        mesh=_sc_mesh(),
        scratch_types=[
            pltpu.VMEM((CPW, 2, KC), jnp.int32),
            pltpu.VMEM((KC, H), jnp.float32),
            pltpu.SemaphoreType.DMA,
            pltpu.SemaphoreType.DMA,
            pltpu.VMEM_SHARED((NP, H), jnp.float32),
        ],
    )


def _dinv_from_deg(deg):
    d = deg[0, :, 0:1] + deg[1, :, 0:1] + 1.0
    return lax.rsqrt(d)


def _gru_hws(x_blk, hprev, dinv, we, be, wih, bih, whh, bhh, wg):
    dn = (((1,), (1,)), ((), ()))
    inp = lax.dot_general(x_blk, we, dn, preferred_element_type=jnp.float32) + be
    gi = lax.dot_general(inp, wih, dn, preferred_element_type=jnp.float32) + bih
    gh = lax.dot_general(hprev, whh, dn, preferred_element_type=jnp.float32) + bhh
    r = jax.nn.sigmoid(gi[:, :H] + gh[:, :H])
    z = jax.nn.sigmoid(gi[:, H:2 * H] + gh[:, H:2 * H])
    cand = jnp.tanh(gi[:, 2 * H:] + r * gh[:, 2 * H:])
    h = (1.0 - z) * cand + z * hprev
    hw = lax.dot_general(h, wg, dn, preferred_element_type=jnp.float32)
    return dinv * hw


_W_SPECS = [
    pl.BlockSpec((H, D_IN), lambda i: (0, 0)),
    pl.BlockSpec((1, H), lambda i: (0, 0)),
    pl.BlockSpec((3 * H, H), lambda i: (0, 0)),
    pl.BlockSpec((1, 3 * H), lambda i: (0, 0)),
    pl.BlockSpec((3 * H, H), lambda i: (0, 0)),
    pl.BlockSpec((1, 3 * H), lambda i: (0, 0)),
    pl.BlockSpec((H, H), lambda i: (0, 0)),
    pl.BlockSpec((1, H), lambda i: (0, 0)),
]


def _make_step0_kernel():
    def body(x_ref, deg_ref, we, be, wih, bih, whh, bhh, wg, bg, out_ref):
        dinv = _dinv_from_deg(deg_ref[...])
        hprev = jnp.zeros((BN, H), jnp.float32)
        out_ref[...] = _gru_hws(x_ref[0], hprev, dinv, we[...], be[...],
                                wih[...], bih[...], whh[...], bhh[...], wg[...])

    in_specs = [
        pl.BlockSpec((1, BN, D_IN), lambda i: (0, i, 0)),
        pl.BlockSpec((NC, BN, H), lambda i: (0, i, 0)),
    ] + _W_SPECS
    return pl.pallas_call(
        body,
        grid=(N // BN,),
        in_specs=in_specs,
        out_specs=pl.BlockSpec((BN, H), lambda i: (i, 0)),
        out_shape=jax.ShapeDtypeStruct((N, H), jnp.float32),
    )


def _make_step_kernel(t):
    def body(x_ref, deg_ref, part_ref, hwsp_ref, we, be, wih, bih, whh, bhh,
             wg, bg, out_ref):
        dinv = _dinv_from_deg(deg_ref[...])
        p = part_ref[...]
        hprev = dinv * (p[0] + p[1] + hwsp_ref[...]) + bg[...]
        out_ref[...] = _gru_hws(x_ref[0], hprev, dinv, we[...], be[...],
                                wih[...], bih[...], whh[...], bhh[...], wg[...])

    in_specs = [
        pl.BlockSpec((1, BN, D_IN), lambda i, _t=t: (_t, i, 0)),
        pl.BlockSpec((NC, BN, H), lambda i: (0, i, 0)),
        pl.BlockSpec((NC, BN, H), lambda i: (0, i, 0)),
        pl.BlockSpec((BN, H), lambda i: (i, 0)),
    ] + _W_SPECS
    return pl.pallas_call(
        body,
        grid=(N // BN,),
        in_specs=in_specs,
        out_specs=pl.BlockSpec((BN, H), lambda i: (i, 0)),
        out_shape=jax.ShapeDtypeStruct((N, H), jnp.float32),
    )


def _make_final_kernel():
    def body(deg_ref, part_ref, hwsp_ref, bg, wfc, bfc, out_ref):
        dinv = _dinv_from_deg(deg_ref[...])
        p = part_ref[...]
        h = dinv * (p[0] + p[1] + hwsp_ref[...]) + bg[...]
        dn = (((1,), (1,)), ((), ()))
        out_ref[...] = lax.dot_general(
            h, wfc[...], dn, preferred_element_type=jnp.float32) + bfc[...]

    in_specs = [
        pl.BlockSpec((NC, BN, H), lambda i: (0, i, 0)),
        pl.BlockSpec((NC, BN, H), lambda i: (0, i, 0)),
        pl.BlockSpec((BN, H), lambda i: (i, 0)),
        pl.BlockSpec((1, H), lambda i: (0, 0)),
        pl.BlockSpec((D_OUT, H), lambda i: (0, 0)),
        pl.BlockSpec((1, D_OUT), lambda i: (0, 0)),
    ]
    return pl.pallas_call(
        body,
        grid=(N // BN,),
        in_specs=in_specs,
        out_specs=pl.BlockSpec((BN, D_OUT), lambda i: (i, 0)),
        out_shape=jax.ShapeDtypeStruct((N, D_OUT), jnp.float32),
    )


def kernel(x, edge_index, W_emb, b_emb, W_ih, W_hh, b_ih, b_hh, W_gcn, b_gcn,
           W_fc, b_fc):
    pad = EPWP - EPW
    padr = jnp.broadcast_to(jnp.arange(pad, dtype=jnp.int32), (NW, pad))
    padc = jnp.broadcast_to(N + jnp.arange(pad, dtype=jnp.int32), (NW, pad))
    rw = jnp.concatenate([edge_index[0].reshape(NW, EPW), padr], axis=1)
    cw = jnp.concatenate([edge_index[1].reshape(NW, EPW), padc], axis=1)
    eidx = jnp.stack([rw.reshape(NW, CPW, KC), cw.reshape(NW, CPW, KC)],
                     axis=2)
    xT = jnp.transpose(x, (1, 0, 2))
    zeros_nh = jnp.zeros((NP, H), jnp.float32)
    ones_nh = jnp.ones((N, H), jnp.float32)
    be = b_emb.reshape(1, H)
    bih = b_ih.reshape(1, 3 * H)
    bhh = b_hh.reshape(1, 3 * H)
    bg = b_gcn.reshape(1, H)
    bfc = b_fc.reshape(1, D_OUT)

    scat = _make_scat_kernel()
    deg = _make_deg_kernel()(eidx, ones_nh, zeros_nh)

    hws = _make_step0_kernel()(
        xT, deg, W_emb, be, W_ih, bih, W_hh, bhh, W_gcn, bg)
    for t in range(T):
        part = scat(hws, eidx, zeros_nh)
        if t < T - 1:
            hws = _make_step_kernel(t + 1)(
                xT, deg, part, hws, W_emb, be, W_ih, bih, W_hh, bhh, W_gcn, bg)
    return _make_final_kernel()(deg, part, hws, bg, W_fc, bfc)


# speedup vs baseline: 1.0070x; 1.0007x over previous
import jax
import jax.numpy as jnp
from jax import lax
from jax.experimental import pallas as pl
from jax.experimental.pallas import tpu as pltpu
from jax.experimental.pallas import tpu_sc as plsc

N = 10000
T = 8
D_IN = 128
H = 128
D_OUT = 128
E = 320000

NC = 2
NS = 16
NW = NC * NS
KC = 128
EPW = E // NW
EPWP = 10240
CPW = EPWP // KC
NP = 10240
RPT = NP // NS
BN = 1000


def _sc_mesh():
    return plsc.VectorSubcoreMesh(core_axis_name="c", subcore_axis_name="s")


Q = 20
NQ = 4


def _scat_body(hws_hbm, eidx_hbm, zeros_hbm, out_hbm,
               ib0, ib1, gb0, gb1, iq0, iq1, gs0, gs1, ps0, ps1, acc_ref):
    cid = lax.axis_index("c")
    sid = lax.axis_index("s")
    wid = cid * NS + sid
    pltpu.sync_copy(zeros_hbm.at[pl.ds(sid * RPT, RPT)],
                    acc_ref.at[pl.ds(sid * RPT, RPT)])
    ibufs = (ib0, ib1)
    isems = (iq0, iq1)
    handles = [
        pltpu.async_copy(eidx_hbm.at[wid, pl.ds(0, Q)], ib0, iq0),
        pltpu.async_copy(eidx_hbm.at[wid, pl.ds(Q, Q)], ib1, iq1),
    ]
    plsc.subcore_barrier()

    def pair(j, drain, _ib):
        if drain:
            pltpu.make_async_copy(zeros_hbm.at[pl.ds(0, KC)], gb0, ps0).wait()
            pltpu.make_async_copy(zeros_hbm.at[pl.ds(0, KC)], gb1, ps1).wait()
        g0 = pltpu.async_copy(hws_hbm.at[_ib.at[j, 0]], gb0, gs0)
        g1 = pltpu.async_copy(hws_hbm.at[_ib.at[j + 1, 0]], gb1, gs1)
        g0.wait()
        pltpu.async_copy(gb0, acc_ref.at[_ib.at[j, 1]], ps0, add=True)
        g1.wait()
        pltpu.async_copy(gb1, acc_ref.at[_ib.at[j + 1, 1]], ps1, add=True)

    for qq in range(NQ):
        ib = ibufs[qq % 2]
        handles[qq].wait()
        pair(0, qq > 0, ib)

        def inner(i, carry, _ib=ib):
            pair(2 * i, True, _ib)
            return carry

        lax.fori_loop(1, Q // 2, inner, 0)
        if qq + 2 < NQ:
            handles.append(pltpu.async_copy(
                eidx_hbm.at[wid, pl.ds((qq + 2) * Q, Q)], ib, isems[qq % 2]))
    pltpu.make_async_copy(zeros_hbm.at[pl.ds(0, KC)], gb0, ps0).wait()
    pltpu.make_async_copy(zeros_hbm.at[pl.ds(0, KC)], gb1, ps1).wait()

    plsc.subcore_barrier()
    pltpu.sync_copy(acc_ref.at[pl.ds(sid * RPT, RPT)],
                    out_hbm.at[cid, pl.ds(sid * RPT, RPT)])


def _make_scat_kernel():
    return pl.kernel(
        _scat_body,
        out_type=jax.ShapeDtypeStruct((NC, NP, H), jnp.float32),
        mesh=_sc_mesh(),
        scratch_types=[
            pltpu.VMEM((Q, 2, KC), jnp.int32),
            pltpu.VMEM((Q, 2, KC), jnp.int32),
            pltpu.VMEM((KC, H), jnp.float32),
            pltpu.VMEM((KC, H), jnp.float32),
            pltpu.SemaphoreType.DMA,
            pltpu.SemaphoreType.DMA,
            pltpu.SemaphoreType.DMA,
            pltpu.SemaphoreType.DMA,
            pltpu.SemaphoreType.DMA,
            pltpu.SemaphoreType.DMA,
            pltpu.VMEM_SHARED((NP, H), jnp.float32),
        ],
    )


def _deg_body(eidx_hbm, ones_hbm, zeros_hbm, out_hbm, idxv, onesv, d0, d1,
              acc_ref):
    cid = lax.axis_index("c")
    sid = lax.axis_index("s")
    wid = cid * NS + sid
    pltpu.sync_copy(eidx_hbm.at[wid], idxv)
    pltpu.sync_copy(ones_hbm.at[pl.ds(0, KC)], onesv)
    pltpu.sync_copy(zeros_hbm.at[pl.ds(sid * RPT, RPT)],
                    acc_ref.at[pl.ds(sid * RPT, RPT)])
    plsc.subcore_barrier()

    def body(q, carry):
        j = 2 * q
        s0 = pltpu.async_copy(onesv, acc_ref.at[idxv.at[j, 1]], d0, add=True)
        s1 = pltpu.async_copy(onesv, acc_ref.at[idxv.at[j + 1, 1]], d1,
                              add=True)
        s0.wait()
        s1.wait()
        return carry

    lax.fori_loop(0, CPW // 2, body, 0)
    plsc.subcore_barrier()
    pltpu.sync_copy(acc_ref.at[pl.ds(sid * RPT, RPT)],
                    out_hbm.at[cid, pl.ds(sid * RPT, RPT)])


def _make_deg_kernel():
    return pl.kernel(
        _deg_body,
        out_type=jax.ShapeDtypeStruct((NC, NP, H), jnp.float32),
        mesh=_sc_mesh(),
        scratch_types=[
            pltpu.VMEM((CPW, 2, KC), jnp.int32),
            pltpu.VMEM((KC, H), jnp.float32),
            pltpu.SemaphoreType.DMA,
            pltpu.SemaphoreType.DMA,
            pltpu.VMEM_SHARED((NP, H), jnp.float32),
        ],
    )


def _dinv_from_deg(deg):
    d = deg[0, :, 0:1] + deg[1, :, 0:1] + 1.0
    return lax.rsqrt(d)


def _gru_hws(x_blk, hprev, dinv, we, be, wih, bih, whh, bhh, wg):
    dn = (((1,), (1,)), ((), ()))
    inp = lax.dot_general(x_blk, we, dn, preferred_element_type=jnp.float32) + be
    gi = lax.dot_general(inp, wih, dn, preferred_element_type=jnp.float32) + bih
    gh = lax.dot_general(hprev, whh, dn, preferred_element_type=jnp.float32) + bhh
    r = jax.nn.sigmoid(gi[:, :H] + gh[:, :H])
    z = jax.nn.sigmoid(gi[:, H:2 * H] + gh[:, H:2 * H])
    cand = jnp.tanh(gi[:, 2 * H:] + r * gh[:, 2 * H:])
    h = (1.0 - z) * cand + z * hprev
    hw = lax.dot_general(h, wg, dn, preferred_element_type=jnp.float32)
    return dinv * hw


_W_SPECS = [
    pl.BlockSpec((H, D_IN), lambda i: (0, 0)),
    pl.BlockSpec((1, H), lambda i: (0, 0)),
    pl.BlockSpec((3 * H, H), lambda i: (0, 0)),
    pl.BlockSpec((1, 3 * H), lambda i: (0, 0)),
    pl.BlockSpec((3 * H, H), lambda i: (0, 0)),
    pl.BlockSpec((1, 3 * H), lambda i: (0, 0)),
    pl.BlockSpec((H, H), lambda i: (0, 0)),
    pl.BlockSpec((1, H), lambda i: (0, 0)),
]


def _make_step0_kernel():
    def body(x_ref, deg_ref, we, be, wih, bih, whh, bhh, wg, bg, out_ref):
        dinv = _dinv_from_deg(deg_ref[...])
        hprev = jnp.zeros((BN, H), jnp.float32)
        out_ref[...] = _gru_hws(x_ref[0], hprev, dinv, we[...], be[...],
                                wih[...], bih[...], whh[...], bhh[...], wg[...])

    in_specs = [
        pl.BlockSpec((1, BN, D_IN), lambda i: (0, i, 0)),
        pl.BlockSpec((NC, BN, H), lambda i: (0, i, 0)),
    ] + _W_SPECS
    return pl.pallas_call(
        body,
        grid=(N // BN,),
        in_specs=in_specs,
        out_specs=pl.BlockSpec((BN, H), lambda i: (i, 0)),
        out_shape=jax.ShapeDtypeStruct((N, H), jnp.float32),
    )


def _make_step_kernel(t):
    def body(x_ref, deg_ref, part_ref, hwsp_ref, we, be, wih, bih, whh, bhh,
             wg, bg, out_ref):
        dinv = _dinv_from_deg(deg_ref[...])
        p = part_ref[...]
        hprev = dinv * (p[0] + p[1] + hwsp_ref[...]) + bg[...]
        out_ref[...] = _gru_hws(x_ref[0], hprev, dinv, we[...], be[...],
                                wih[...], bih[...], whh[...], bhh[...], wg[...])

    in_specs = [
        pl.BlockSpec((1, BN, D_IN), lambda i, _t=t: (_t, i, 0)),
        pl.BlockSpec((NC, BN, H), lambda i: (0, i, 0)),
        pl.BlockSpec((NC, BN, H), lambda i: (0, i, 0)),
        pl.BlockSpec((BN, H), lambda i: (i, 0)),
    ] + _W_SPECS
    return pl.pallas_call(
        body,
        grid=(N // BN,),
        in_specs=in_specs,
        out_specs=pl.BlockSpec((BN, H), lambda i: (i, 0)),
        out_shape=jax.ShapeDtypeStruct((N, H), jnp.float32),
    )


def _make_final_kernel():
    def body(deg_ref, part_ref, hwsp_ref, bg, wfc, bfc, out_ref):
        dinv = _dinv_from_deg(deg_ref[...])
        p = part_ref[...]
        h = dinv * (p[0] + p[1] + hwsp_ref[...]) + bg[...]
        dn = (((1,), (1,)), ((), ()))
        out_ref[...] = lax.dot_general(
            h, wfc[...], dn, preferred_element_type=jnp.float32) + bfc[...]

    in_specs = [
        pl.BlockSpec((NC, BN, H), lambda i: (0, i, 0)),
        pl.BlockSpec((NC, BN, H), lambda i: (0, i, 0)),
        pl.BlockSpec((BN, H), lambda i: (i, 0)),
        pl.BlockSpec((1, H), lambda i: (0, 0)),
        pl.BlockSpec((D_OUT, H), lambda i: (0, 0)),
        pl.BlockSpec((1, D_OUT), lambda i: (0, 0)),
    ]
    return pl.pallas_call(
        body,
        grid=(N // BN,),
        in_specs=in_specs,
        out_specs=pl.BlockSpec((BN, D_OUT), lambda i: (i, 0)),
        out_shape=jax.ShapeDtypeStruct((N, D_OUT), jnp.float32),
    )


def kernel(x, edge_index, W_emb, b_emb, W_ih, W_hh, b_ih, b_hh, W_gcn, b_gcn,
           W_fc, b_fc):
    pad = EPWP - EPW
    padr = jnp.broadcast_to(jnp.arange(pad, dtype=jnp.int32), (NW, pad))
    padc = jnp.broadcast_to(N + jnp.arange(pad, dtype=jnp.int32), (NW, pad))
    rw = jnp.concatenate([edge_index[0].reshape(NW, EPW), padr], axis=1)
    cw = jnp.concatenate([edge_index[1].reshape(NW, EPW), padc], axis=1)
    eidx = jnp.stack([rw.reshape(NW, CPW, KC), cw.reshape(NW, CPW, KC)],
                     axis=2)
    xT = jnp.transpose(x, (1, 0, 2))
    zeros_nh = jnp.zeros((NP, H), jnp.float32)
    ones_nh = jnp.ones((N, H), jnp.float32)
    be = b_emb.reshape(1, H)
    bih = b_ih.reshape(1, 3 * H)
    bhh = b_hh.reshape(1, 3 * H)
    bg = b_gcn.reshape(1, H)
    bfc = b_fc.reshape(1, D_OUT)

    scat = _make_scat_kernel()
    deg = _make_deg_kernel()(eidx, ones_nh, zeros_nh)

    hws = _make_step0_kernel()(
        xT, deg, W_emb, be, W_ih, bih, W_hh, bhh, W_gcn, bg)
    for t in range(T):
        part = scat(hws, eidx, zeros_nh)
        if t < T - 1:
            hws = _make_step_kernel(t + 1)(
                xT, deg, part, hws, W_emb, be, W_ih, bih, W_hh, bhh, W_gcn, bg)
    return _make_final_kernel()(deg, part, hws, bg, W_fc, bfc)
